# baseline (device time: 135863 ns/iter reference)
import jax
import jax.numpy as jnp
from jax import lax
from jax.experimental import pallas as pl
from jax.experimental.pallas import tpu as pltpu

N_DEV = 4
N_SPLIT = 2
SCALE = 0.08838834764831843
SCALE2 = SCALE * 1.4426950408889634


def kernel(x, Wq, Wo, K_ext, V_ext):
    _, Sq, D = x.shape
    _, Skv, Hq, Dh = K_ext.shape
    rows = Sq // N_SPLIT

    x2 = x.reshape(Sq, D).astype(jnp.bfloat16)
    Wq2 = Wq.astype(jnp.bfloat16)
    Wo2 = Wo.astype(jnp.bfloat16)
    K2 = K_ext.reshape(Skv, Hq * Dh)
    V2 = V_ext.reshape(Skv, Hq * Dh)

    def body(x_ref, wq_ref, wo_ref, k_ref, v_ref, out_ref,
             q_buf, acc_buf, ml_buf, acc_stage, ml_stage,
             q_ssem, q_rsem, a_ssem, a_rsem, m_ssem, m_rsem):
        my = lax.axis_index("i")
        left = lax.rem(my + N_DEV - 1, N_DEV)
        right = lax.rem(my + 1, N_DEV)

        barrier_sem = pltpu.get_barrier_semaphore()
        for nbr in (left, right):
            pl.semaphore_signal(
                barrier_sem, inc=1,
                device_id=(nbr,), device_id_type=pl.DeviceIdType.MESH,
            )
        pl.semaphore_wait(barrier_sem, 2)

        q0 = lax.dot_general(
            x_ref[:, :], wq_ref[:, :], (((1,), (0,)), ((), ())),
            preferred_element_type=jnp.float32,
        )
        q_buf[0] = (q0 * SCALE2).astype(jnp.bfloat16)

        q_rdmas = [None] * (N_DEV - 1)
        a_rdmas = {}
        m_rdmas = {}
        sends_unwaited = {}

        for h in range(N_DEV):
            if h > 0:
                q_rdmas[h - 1].wait_recv()
            if h < N_DEV - 1:
                rq = pltpu.make_async_remote_copy(
                    src_ref=q_buf.at[h], dst_ref=q_buf.at[h + 1],
                    send_sem=q_ssem.at[h], recv_sem=q_rsem.at[h],
                    device_id=(right,), device_id_type=pl.DeviceIdType.MESH,
                )
                rq.start()
                q_rdmas[h] = rq
                sends_unwaited[("q", h)] = rq

            for half in range(N_SPLIT):
                if h > 0:
                    a_rdmas[(h - 1, half)].wait_recv()
                    m_rdmas[(h - 1, half)].wait_recv()
                if h >= 2:
                    sends_unwaited.pop(("a", h - 2, half)).wait_send()
                    sends_unwaited.pop(("m", h - 2, half)).wait_send()
                rs = slice(half * rows, (half + 1) * rows)
                m_cols = []
                l_cols = []
                for i in range(Hq):
                    sl = slice(i * Dh, (i + 1) * Dh)
                    q_i = q_buf[h, rs, sl]
                    k_i = k_ref[:, sl].astype(jnp.bfloat16)
                    v_i = v_ref[:, sl].astype(jnp.bfloat16)
                    s = lax.dot_general(
                        q_i, k_i, (((1,), (1,)), ((), ())),
                        preferred_element_type=jnp.float32,
                    )
                    mj = jnp.max(s, axis=1, keepdims=True)
                    if h == 0:
                        m_new = mj
                        p = jnp.exp2(s - m_new)
                        l_new = jnp.sum(p, axis=1, keepdims=True)
                        acc_new = lax.dot_general(
                            p.astype(jnp.bfloat16), v_i,
                            (((1,), (0,)), ((), ())),
                            preferred_element_type=jnp.float32,
                        )
                    else:
                        m_old = ml_buf[h - 1, rs, i:i + 1]
                        l_old = ml_buf[h - 1, rs, Hq + i:Hq + i + 1]
                        m_new = jnp.maximum(m_old, mj)
                        alpha = jnp.exp2(m_old - m_new)
                        p = jnp.exp2(s - m_new)
                        l_new = l_old * alpha + jnp.sum(p, axis=1,
                                                        keepdims=True)
                        acc_new = acc_buf[h - 1, rs, sl] * alpha + \
                            lax.dot_general(
                                p.astype(jnp.bfloat16), v_i,
                                (((1,), (0,)), ((), ())),
                                preferred_element_type=jnp.float32,
                            )
                    m_cols.append(m_new)
                    l_cols.append(l_new)
                    acc_stage[h % 2, rs, sl] = acc_new.astype(jnp.bfloat16)

                ml_stage[h % 2, rs, :] = jnp.concatenate(m_cols + l_cols,
                                                         axis=1)

                r0 = half * rows
                ra = pltpu.make_async_remote_copy(
                    src_ref=acc_stage.at[h % 2, pl.ds(r0, rows)],
                    dst_ref=acc_buf.at[h, pl.ds(r0, rows)],
                    send_sem=a_ssem.at[h, half], recv_sem=a_rsem.at[h, half],
                    device_id=(right,), device_id_type=pl.DeviceIdType.MESH,
                )
                rm = pltpu.make_async_remote_copy(
                    src_ref=ml_stage.at[h % 2, pl.ds(r0, rows)],
                    dst_ref=ml_buf.at[h, pl.ds(r0, rows)],
                    send_sem=m_ssem.at[h, half], recv_sem=m_rsem.at[h, half],
                    device_id=(right,), device_id_type=pl.DeviceIdType.MESH,
                )
                ra.start()
                rm.start()
                a_rdmas[(h, half)] = ra
                m_rdmas[(h, half)] = rm
                sends_unwaited[("a", h, half)] = ra
                sends_unwaited[("m", h, half)] = rm

        for half in range(N_SPLIT):
            a_rdmas[(N_DEV - 1, half)].wait_recv()
            m_rdmas[(N_DEV - 1, half)].wait_recv()
        for r in sends_unwaited.values():
            r.wait_send()
        outs = []
        for i in range(Hq):
            sl = slice(i * Dh, (i + 1) * Dh)
            l_i = ml_buf[N_DEV - 1, :, Hq + i:Hq + i + 1]
            outs.append((acc_buf[N_DEV - 1, :, sl] / l_i).astype(jnp.bfloat16))
        o = jnp.concatenate(outs, axis=1)
        out_ref[:, :] = lax.dot_general(
            o, wo_ref[:, :], (((1,), (0,)), ((), ())),
            preferred_element_type=jnp.float32,
        )

    out = pl.pallas_call(
        body,
        out_shape=jax.ShapeDtypeStruct((Sq, D), jnp.float32),
        in_specs=[pl.BlockSpec(memory_space=pltpu.VMEM)] * 5,
        out_specs=pl.BlockSpec(memory_space=pltpu.VMEM),
        scratch_shapes=[
            pltpu.VMEM((N_DEV, Sq, Hq * Dh), jnp.bfloat16),
            pltpu.VMEM((N_DEV, Sq, Hq * Dh), jnp.bfloat16),
            pltpu.VMEM((N_DEV, Sq, 2 * Hq), jnp.float32),
            pltpu.VMEM((2, Sq, Hq * Dh), jnp.bfloat16),
            pltpu.VMEM((2, Sq, 2 * Hq), jnp.float32),
            pltpu.SemaphoreType.DMA((N_DEV - 1,)),
            pltpu.SemaphoreType.DMA((N_DEV - 1,)),
            pltpu.SemaphoreType.DMA((N_DEV, N_SPLIT)),
            pltpu.SemaphoreType.DMA((N_DEV, N_SPLIT)),
            pltpu.SemaphoreType.DMA((N_DEV, N_SPLIT)),
            pltpu.SemaphoreType.DMA((N_DEV, N_SPLIT)),
        ],
        compiler_params=pltpu.CompilerParams(
            collective_id=0,
            vmem_limit_bytes=100 * 1024 * 1024,
        ),
    )(x2, Wq2, Wo2, K2, V2)
    return out.reshape(1, Sq, D)


# device time: 129247 ns/iter; 1.0512x vs baseline; 1.0512x over previous
import jax
import jax.numpy as jnp
from jax import lax
from jax.experimental import pallas as pl
from jax.experimental.pallas import tpu as pltpu

N_DEV = 4
N_SPLIT = 2
SCALE = 0.08838834764831843
SCALE2 = SCALE * 1.4426950408889634


def kernel(x, Wq, Wo, K_ext, V_ext):
    _, Sq, D = x.shape
    _, Skv, Hq, Dh = K_ext.shape
    rows = Sq // N_SPLIT

    x2 = x.reshape(Sq, D).astype(jnp.bfloat16)
    Wq2 = Wq.astype(jnp.bfloat16)
    Wo2 = Wo.astype(jnp.bfloat16)
    K2 = K_ext.reshape(Skv, Hq * Dh).astype(jnp.bfloat16)
    V2 = V_ext.reshape(Skv, Hq * Dh).astype(jnp.bfloat16)

    def body(x_ref, wq_ref, wo_ref, k_ref, v_ref, out_ref,
             q_buf, acc_buf, ml_buf, acc_stage, ml_stage,
             q_ssem, q_rsem, a_ssem, a_rsem, m_ssem, m_rsem):
        my = lax.axis_index("i")
        left = lax.rem(my + N_DEV - 1, N_DEV)
        right = lax.rem(my + 1, N_DEV)

        barrier_sem = pltpu.get_barrier_semaphore()
        for nbr in (left, right):
            pl.semaphore_signal(
                barrier_sem, inc=1,
                device_id=(nbr,), device_id_type=pl.DeviceIdType.MESH,
            )
        pl.semaphore_wait(barrier_sem, 2)

        q0 = lax.dot_general(
            x_ref[:, :], wq_ref[:, :], (((1,), (0,)), ((), ())),
            preferred_element_type=jnp.float32,
        )
        q_buf[0] = (q0 * SCALE2).astype(jnp.bfloat16)

        q_rdmas = [None] * (N_DEV - 1)
        a_rdmas = {}
        m_rdmas = {}
        sends_unwaited = {}

        for h in range(N_DEV):
            if h > 0:
                q_rdmas[h - 1].wait_recv()
            if h < N_DEV - 1:
                rq = pltpu.make_async_remote_copy(
                    src_ref=q_buf.at[h], dst_ref=q_buf.at[h + 1],
                    send_sem=q_ssem.at[h], recv_sem=q_rsem.at[h],
                    device_id=(right,), device_id_type=pl.DeviceIdType.MESH,
                )
                rq.start()
                q_rdmas[h] = rq
                sends_unwaited[("q", h)] = rq

            for half in range(N_SPLIT):
                if h > 0:
                    a_rdmas[(h - 1, half)].wait_recv()
                    m_rdmas[(h - 1, half)].wait_recv()
                if h >= 2:
                    sends_unwaited.pop(("a", h - 2, half)).wait_send()
                    sends_unwaited.pop(("m", h - 2, half)).wait_send()
                rs = slice(half * rows, (half + 1) * rows)
                m_cols = []
                l_cols = []
                for i in range(Hq):
                    sl = slice(i * Dh, (i + 1) * Dh)
                    q_i = q_buf[h, rs, sl]
                    k_i = k_ref[:, sl]
                    v_i = v_ref[:, sl]
                    s = lax.dot_general(
                        q_i, k_i, (((1,), (1,)), ((), ())),
                        preferred_element_type=jnp.float32,
                    )
                    mj = jnp.max(s, axis=1, keepdims=True)
                    if h == 0:
                        m_new = mj
                        p = jnp.exp2(s - m_new)
                        l_new = jnp.sum(p, axis=1, keepdims=True)
                        acc_new = lax.dot_general(
                            p.astype(jnp.bfloat16), v_i,
                            (((1,), (0,)), ((), ())),
                            preferred_element_type=jnp.float32,
                        )
                    else:
                        m_old = ml_buf[h - 1, rs, i:i + 1]
                        l_old = ml_buf[h - 1, rs, Hq + i:Hq + i + 1]
                        m_new = jnp.maximum(m_old, mj)
                        alpha = jnp.exp2(m_old - m_new)
                        p = jnp.exp2(s - m_new)
                        l_new = l_old * alpha + jnp.sum(p, axis=1,
                                                        keepdims=True)
                        acc_new = acc_buf[h - 1, rs, sl] * alpha + \
                            lax.dot_general(
                                p.astype(jnp.bfloat16), v_i,
                                (((1,), (0,)), ((), ())),
                                preferred_element_type=jnp.float32,
                            )
                    m_cols.append(m_new)
                    l_cols.append(l_new)
                    acc_stage[h % 2, rs, sl] = acc_new.astype(jnp.bfloat16)

                ml_stage[h % 2, rs, :] = jnp.concatenate(m_cols + l_cols,
                                                         axis=1)

                r0 = half * rows
                ra = pltpu.make_async_remote_copy(
                    src_ref=acc_stage.at[h % 2, pl.ds(r0, rows)],
                    dst_ref=acc_buf.at[h, pl.ds(r0, rows)],
                    send_sem=a_ssem.at[h, half], recv_sem=a_rsem.at[h, half],
                    device_id=(right,), device_id_type=pl.DeviceIdType.MESH,
                )
                rm = pltpu.make_async_remote_copy(
                    src_ref=ml_stage.at[h % 2, pl.ds(r0, rows)],
                    dst_ref=ml_buf.at[h, pl.ds(r0, rows)],
                    send_sem=m_ssem.at[h, half], recv_sem=m_rsem.at[h, half],
                    device_id=(right,), device_id_type=pl.DeviceIdType.MESH,
                )
                ra.start()
                rm.start()
                a_rdmas[(h, half)] = ra
                m_rdmas[(h, half)] = rm
                sends_unwaited[("a", h, half)] = ra
                sends_unwaited[("m", h, half)] = rm

        for half in range(N_SPLIT):
            a_rdmas[(N_DEV - 1, half)].wait_recv()
            m_rdmas[(N_DEV - 1, half)].wait_recv()
        for r in sends_unwaited.values():
            r.wait_send()
        outs = []
        for i in range(Hq):
            sl = slice(i * Dh, (i + 1) * Dh)
            l_i = ml_buf[N_DEV - 1, :, Hq + i:Hq + i + 1]
            outs.append((acc_buf[N_DEV - 1, :, sl] / l_i).astype(jnp.bfloat16))
        o = jnp.concatenate(outs, axis=1)
        out_ref[:, :] = lax.dot_general(
            o, wo_ref[:, :], (((1,), (0,)), ((), ())),
            preferred_element_type=jnp.float32,
        )

    out = pl.pallas_call(
        body,
        out_shape=jax.ShapeDtypeStruct((Sq, D), jnp.float32),
        in_specs=[pl.BlockSpec(memory_space=pltpu.VMEM)] * 5,
        out_specs=pl.BlockSpec(memory_space=pltpu.VMEM),
        scratch_shapes=[
            pltpu.VMEM((N_DEV, Sq, Hq * Dh), jnp.bfloat16),
            pltpu.VMEM((N_DEV, Sq, Hq * Dh), jnp.bfloat16),
            pltpu.VMEM((N_DEV, Sq, 2 * Hq), jnp.float32),
            pltpu.VMEM((2, Sq, Hq * Dh), jnp.bfloat16),
            pltpu.VMEM((2, Sq, 2 * Hq), jnp.float32),
            pltpu.SemaphoreType.DMA((N_DEV - 1,)),
            pltpu.SemaphoreType.DMA((N_DEV - 1,)),
            pltpu.SemaphoreType.DMA((N_DEV, N_SPLIT)),
            pltpu.SemaphoreType.DMA((N_DEV, N_SPLIT)),
            pltpu.SemaphoreType.DMA((N_DEV, N_SPLIT)),
            pltpu.SemaphoreType.DMA((N_DEV, N_SPLIT)),
        ],
        compiler_params=pltpu.CompilerParams(
            collective_id=0,
            vmem_limit_bytes=100 * 1024 * 1024,
        ),
    )(x2, Wq2, Wo2, K2, V2)
    return out.reshape(1, Sq, D)


# device time: 122940 ns/iter; 1.1051x vs baseline; 1.0513x over previous
import jax
import jax.numpy as jnp
from jax import lax
from jax.experimental import pallas as pl
from jax.experimental.pallas import tpu as pltpu

N_DEV = 4
N_SPLIT = 2
SCALE = 0.08838834764831843
SCALE2 = SCALE * 1.4426950408889634


def kernel(x, Wq, Wo, K_ext, V_ext):
    _, Sq, D = x.shape
    _, Skv, Hq, Dh = K_ext.shape
    rows = Sq // N_SPLIT

    x2 = x.reshape(Sq, D)
    K2 = K_ext.reshape(Skv, Hq * Dh).astype(jnp.bfloat16)
    V2 = V_ext.reshape(Skv, Hq * Dh).astype(jnp.bfloat16)

    def body(x_ref, wq_ref, wo_ref, k_ref, v_ref, out_ref,
             q_buf, acc_buf, ml_buf, acc_stage, ml_stage,
             q_ssem, q_rsem, a_ssem, a_rsem, m_ssem, m_rsem):
        my = lax.axis_index("i")
        left = lax.rem(my + N_DEV - 1, N_DEV)
        right = lax.rem(my + 1, N_DEV)

        barrier_sem = pltpu.get_barrier_semaphore()
        for nbr in (left, right):
            pl.semaphore_signal(
                barrier_sem, inc=1,
                device_id=(nbr,), device_id_type=pl.DeviceIdType.MESH,
            )
        pl.semaphore_wait(barrier_sem, 2)

        q0 = lax.dot_general(
            x_ref[:, :].astype(jnp.bfloat16),
            wq_ref[:, :].astype(jnp.bfloat16),
            (((1,), (0,)), ((), ())),
            preferred_element_type=jnp.float32,
        )
        q_buf[0] = (q0 * SCALE2).astype(jnp.bfloat16)

        q_rdmas = [None] * (N_DEV - 1)
        a_rdmas = {}
        m_rdmas = {}
        sends_unwaited = {}

        for h in range(N_DEV):
            if h > 0:
                q_rdmas[h - 1].wait_recv()
            if h < N_DEV - 1:
                rq = pltpu.make_async_remote_copy(
                    src_ref=q_buf.at[h], dst_ref=q_buf.at[h + 1],
                    send_sem=q_ssem.at[h], recv_sem=q_rsem.at[h],
                    device_id=(right,), device_id_type=pl.DeviceIdType.MESH,
                )
                rq.start()
                q_rdmas[h] = rq
                sends_unwaited[("q", h)] = rq

            for half in range(N_SPLIT):
                if h > 0:
                    a_rdmas[(h - 1, half)].wait_recv()
                    m_rdmas[(h - 1, half)].wait_recv()
                if h >= 2:
                    sends_unwaited.pop(("a", h - 2, half)).wait_send()
                    sends_unwaited.pop(("m", h - 2, half)).wait_send()
                rs = slice(half * rows, (half + 1) * rows)
                m_cols = []
                l_cols = []
                for i in range(Hq):
                    sl = slice(i * Dh, (i + 1) * Dh)
                    q_i = q_buf[h, rs, sl]
                    k_i = k_ref[:, sl]
                    v_i = v_ref[:, sl]
                    s = lax.dot_general(
                        q_i, k_i, (((1,), (1,)), ((), ())),
                        preferred_element_type=jnp.float32,
                    )
                    mj = jnp.max(s, axis=1, keepdims=True)
                    if h == 0:
                        m_new = mj
                        p = jnp.exp2(s - m_new)
                        l_new = jnp.sum(p, axis=1, keepdims=True)
                        acc_new = lax.dot_general(
                            p.astype(jnp.bfloat16), v_i,
                            (((1,), (0,)), ((), ())),
                            preferred_element_type=jnp.float32,
                        )
                    else:
                        m_old = ml_buf[h - 1, rs, i:i + 1]
                        l_old = ml_buf[h - 1, rs, Hq + i:Hq + i + 1]
                        m_new = jnp.maximum(m_old, mj)
                        alpha = jnp.exp2(m_old - m_new)
                        p = jnp.exp2(s - m_new)
                        l_new = l_old * alpha + jnp.sum(p, axis=1,
                                                        keepdims=True)
                        acc_new = acc_buf[h - 1, rs, sl] * alpha + \
                            lax.dot_general(
                                p.astype(jnp.bfloat16), v_i,
                                (((1,), (0,)), ((), ())),
                                preferred_element_type=jnp.float32,
                            )
                    m_cols.append(m_new)
                    l_cols.append(l_new)
                    acc_stage[h % 2, rs, sl] = acc_new.astype(jnp.bfloat16)

                ml_stage[h % 2, rs, :] = jnp.concatenate(m_cols + l_cols,
                                                         axis=1)

                r0 = half * rows
                ra = pltpu.make_async_remote_copy(
                    src_ref=acc_stage.at[h % 2, pl.ds(r0, rows)],
                    dst_ref=acc_buf.at[h, pl.ds(r0, rows)],
                    send_sem=a_ssem.at[h, half], recv_sem=a_rsem.at[h, half],
                    device_id=(right,), device_id_type=pl.DeviceIdType.MESH,
                )
                rm = pltpu.make_async_remote_copy(
                    src_ref=ml_stage.at[h % 2, pl.ds(r0, rows)],
                    dst_ref=ml_buf.at[h, pl.ds(r0, rows)],
                    send_sem=m_ssem.at[h, half], recv_sem=m_rsem.at[h, half],
                    device_id=(right,), device_id_type=pl.DeviceIdType.MESH,
                )
                ra.start()
                rm.start()
                a_rdmas[(h, half)] = ra
                m_rdmas[(h, half)] = rm
                sends_unwaited[("a", h, half)] = ra
                sends_unwaited[("m", h, half)] = rm

        wo_bf = wo_ref[:, :].astype(jnp.bfloat16)
        for half in range(N_SPLIT):
            rs = slice(half * rows, (half + 1) * rows)
            a_rdmas[(N_DEV - 1, half)].wait_recv()
            m_rdmas[(N_DEV - 1, half)].wait_recv()
            outs = []
            for i in range(Hq):
                sl = slice(i * Dh, (i + 1) * Dh)
                l_i = ml_buf[N_DEV - 1, rs, Hq + i:Hq + i + 1]
                outs.append(
                    (acc_buf[N_DEV - 1, rs, sl] / l_i).astype(jnp.bfloat16))
            o = jnp.concatenate(outs, axis=1)
            out_ref[rs, :] = lax.dot_general(
                o, wo_bf, (((1,), (0,)), ((), ())),
                preferred_element_type=jnp.float32,
            ).astype(jnp.bfloat16)
        for r in sends_unwaited.values():
            r.wait_send()

    out = pl.pallas_call(
        body,
        out_shape=jax.ShapeDtypeStruct((Sq, D), jnp.bfloat16),
        in_specs=[pl.BlockSpec(memory_space=pltpu.VMEM)] * 5,
        out_specs=pl.BlockSpec(memory_space=pltpu.VMEM),
        scratch_shapes=[
            pltpu.VMEM((N_DEV, Sq, Hq * Dh), jnp.bfloat16),
            pltpu.VMEM((N_DEV, Sq, Hq * Dh), jnp.bfloat16),
            pltpu.VMEM((N_DEV, Sq, 2 * Hq), jnp.float32),
            pltpu.VMEM((2, Sq, Hq * Dh), jnp.bfloat16),
            pltpu.VMEM((2, Sq, 2 * Hq), jnp.float32),
            pltpu.SemaphoreType.DMA((N_DEV - 1,)),
            pltpu.SemaphoreType.DMA((N_DEV - 1,)),
            pltpu.SemaphoreType.DMA((N_DEV, N_SPLIT)),
            pltpu.SemaphoreType.DMA((N_DEV, N_SPLIT)),
            pltpu.SemaphoreType.DMA((N_DEV, N_SPLIT)),
            pltpu.SemaphoreType.DMA((N_DEV, N_SPLIT)),
        ],
        compiler_params=pltpu.CompilerParams(
            collective_id=0,
            vmem_limit_bytes=100 * 1024 * 1024,
        ),
    )(x2, Wq, Wo, K2, V2)
    return out.reshape(1, Sq, D)


# device time: 120289 ns/iter; 1.1295x vs baseline; 1.0220x over previous
import jax
import jax.numpy as jnp
from jax import lax
from jax.experimental import pallas as pl
from jax.experimental.pallas import tpu as pltpu

N_DEV = 4
N_SPLIT = 2
SCALE = 0.08838834764831843
SCALE2 = SCALE * 1.4426950408889634


def kernel(x, Wq, Wo, K_ext, V_ext):
    _, Sq, D = x.shape
    _, Skv, Hq, Dh = K_ext.shape
    rows = Sq // N_SPLIT

    x2 = x.reshape(Sq, D)
    K2 = K_ext.reshape(Skv, Hq * Dh).astype(jnp.bfloat16)
    V2 = V_ext.reshape(Skv, Hq * Dh).astype(jnp.bfloat16)

    def body(x_ref, wq_ref, wo_ref, k_ref, v_ref, out_ref,
             q_buf, acc_buf, ml_buf, acc_stage, ml_stage, va_ref,
             q_ssem, q_rsem, a_ssem, a_rsem, m_ssem, m_rsem):
        my = lax.axis_index("i")
        left = lax.rem(my + N_DEV - 1, N_DEV)
        right = lax.rem(my + 1, N_DEV)

        barrier_sem = pltpu.get_barrier_semaphore()
        for nbr in (left, right):
            pl.semaphore_signal(
                barrier_sem, inc=1,
                device_id=(nbr,), device_id_type=pl.DeviceIdType.MESH,
            )
        pl.semaphore_wait(barrier_sem, 2)

        for i in range(Hq):
            va_ref[i, :, 0:Dh] = v_ref[:, i * Dh:(i + 1) * Dh]
            va_ref[i, :, Dh:Dh + 1] = jnp.ones((Skv, 1), jnp.bfloat16)

        q0 = lax.dot_general(
            x_ref[:, :].astype(jnp.bfloat16),
            wq_ref[:, :].astype(jnp.bfloat16),
            (((1,), (0,)), ((), ())),
            preferred_element_type=jnp.float32,
        )
        q_buf[0] = (q0 * SCALE2).astype(jnp.bfloat16)

        q_rdmas = [None] * (N_DEV - 1)
        a_rdmas = {}
        m_rdmas = {}
        sends_unwaited = {}

        for h in range(N_DEV):
            if h > 0:
                q_rdmas[h - 1].wait_recv()
            if h < N_DEV - 1:
                rq = pltpu.make_async_remote_copy(
                    src_ref=q_buf.at[h], dst_ref=q_buf.at[h + 1],
                    send_sem=q_ssem.at[h], recv_sem=q_rsem.at[h],
                    device_id=(right,), device_id_type=pl.DeviceIdType.MESH,
                )
                rq.start()
                q_rdmas[h] = rq
                sends_unwaited[("q", h)] = rq

            for half in range(N_SPLIT):
                if h > 0:
                    a_rdmas[(h - 1, half)].wait_recv()
                    m_rdmas[(h - 1, half)].wait_recv()
                if h >= 2:
                    sends_unwaited.pop(("a", h - 2, half)).wait_send()
                    sends_unwaited.pop(("m", h - 2, half)).wait_send()
                rs = slice(half * rows, (half + 1) * rows)
                m_cols = []
                l_cols = []
                for i in range(Hq):
                    sl = slice(i * Dh, (i + 1) * Dh)
                    q_i = q_buf[h, rs, sl]
                    k_i = k_ref[:, sl]
                    s = lax.dot_general(
                        q_i, k_i, (((1,), (1,)), ((), ())),
                        preferred_element_type=jnp.float32,
                    )
                    mj = jnp.max(s, axis=1, keepdims=True)
                    if h == 0:
                        m_new = mj
                        p = jnp.exp2(s - m_new)
                        pv = lax.dot_general(
                            p.astype(jnp.bfloat16), va_ref[i],
                            (((1,), (0,)), ((), ())),
                            preferred_element_type=jnp.float32,
                        )
                        l_new = pv[:, Dh:Dh + 1]
                        acc_new = pv[:, 0:Dh]
                    else:
                        m_old = ml_buf[h - 1, rs, i:i + 1]
                        l_old = ml_buf[h - 1, rs, Hq + i:Hq + i + 1]
                        m_new = jnp.maximum(m_old, mj)
                        alpha = jnp.exp2(m_old - m_new)
                        p = jnp.exp2(s - m_new)
                        pv = lax.dot_general(
                            p.astype(jnp.bfloat16), va_ref[i],
                            (((1,), (0,)), ((), ())),
                            preferred_element_type=jnp.float32,
                        )
                        l_new = l_old * alpha + pv[:, Dh:Dh + 1]
                        acc_new = acc_buf[h - 1, rs, sl] * alpha + pv[:, 0:Dh]
                    m_cols.append(m_new)
                    l_cols.append(l_new)
                    acc_stage[h % 2, rs, sl] = acc_new.astype(jnp.bfloat16)

                ml_stage[h % 2, rs, :] = jnp.concatenate(m_cols + l_cols,
                                                         axis=1)

                r0 = half * rows
                ra = pltpu.make_async_remote_copy(
                    src_ref=acc_stage.at[h % 2, pl.ds(r0, rows)],
                    dst_ref=acc_buf.at[h, pl.ds(r0, rows)],
                    send_sem=a_ssem.at[h, half], recv_sem=a_rsem.at[h, half],
                    device_id=(right,), device_id_type=pl.DeviceIdType.MESH,
                )
                rm = pltpu.make_async_remote_copy(
                    src_ref=ml_stage.at[h % 2, pl.ds(r0, rows)],
                    dst_ref=ml_buf.at[h, pl.ds(r0, rows)],
                    send_sem=m_ssem.at[h, half], recv_sem=m_rsem.at[h, half],
                    device_id=(right,), device_id_type=pl.DeviceIdType.MESH,
                )
                ra.start()
                rm.start()
                a_rdmas[(h, half)] = ra
                m_rdmas[(h, half)] = rm
                sends_unwaited[("a", h, half)] = ra
                sends_unwaited[("m", h, half)] = rm

        wo_bf = wo_ref[:, :].astype(jnp.bfloat16)
        for half in range(N_SPLIT):
            rs = slice(half * rows, (half + 1) * rows)
            a_rdmas[(N_DEV - 1, half)].wait_recv()
            m_rdmas[(N_DEV - 1, half)].wait_recv()
            outs = []
            for i in range(Hq):
                sl = slice(i * Dh, (i + 1) * Dh)
                l_i = ml_buf[N_DEV - 1, rs, Hq + i:Hq + i + 1]
                outs.append(
                    (acc_buf[N_DEV - 1, rs, sl] / l_i).astype(jnp.bfloat16))
            o = jnp.concatenate(outs, axis=1)
            out_ref[rs, :] = lax.dot_general(
                o, wo_bf, (((1,), (0,)), ((), ())),
                preferred_element_type=jnp.float32,
            ).astype(jnp.bfloat16)
        for r in sends_unwaited.values():
            r.wait_send()

    out = pl.pallas_call(
        body,
        out_shape=jax.ShapeDtypeStruct((Sq, D), jnp.bfloat16),
        in_specs=[pl.BlockSpec(memory_space=pltpu.VMEM)] * 5,
        out_specs=pl.BlockSpec(memory_space=pltpu.VMEM),
        scratch_shapes=[
            pltpu.VMEM((N_DEV, Sq, Hq * Dh), jnp.bfloat16),
            pltpu.VMEM((N_DEV, Sq, Hq * Dh), jnp.bfloat16),
            pltpu.VMEM((N_DEV, Sq, 2 * Hq), jnp.float32),
            pltpu.VMEM((2, Sq, Hq * Dh), jnp.bfloat16),
            pltpu.VMEM((2, Sq, 2 * Hq), jnp.float32),
            pltpu.VMEM((Hq, Skv, Dh + 1), jnp.bfloat16),
            pltpu.SemaphoreType.DMA((N_DEV - 1,)),
            pltpu.SemaphoreType.DMA((N_DEV - 1,)),
            pltpu.SemaphoreType.DMA((N_DEV, N_SPLIT)),
            pltpu.SemaphoreType.DMA((N_DEV, N_SPLIT)),
            pltpu.SemaphoreType.DMA((N_DEV, N_SPLIT)),
            pltpu.SemaphoreType.DMA((N_DEV, N_SPLIT)),
        ],
        compiler_params=pltpu.CompilerParams(
            collective_id=0,
            vmem_limit_bytes=100 * 1024 * 1024,
        ),
    )(x2, Wq, Wo, K2, V2)
    return out.reshape(1, Sq, D)


# device time: 119887 ns/iter; 1.1333x vs baseline; 1.0034x over previous
import jax
import jax.numpy as jnp
from jax import lax
from jax.experimental import pallas as pl
from jax.experimental.pallas import tpu as pltpu

N_DEV = 4
N_SPLIT = 4
SCALE = 0.08838834764831843
SCALE2 = SCALE * 1.4426950408889634


def kernel(x, Wq, Wo, K_ext, V_ext):
    _, Sq, D = x.shape
    _, Skv, Hq, Dh = K_ext.shape
    rows = Sq // N_SPLIT

    x2 = x.reshape(Sq, D)
    K2 = K_ext.reshape(Skv, Hq * Dh).astype(jnp.bfloat16)
    V2 = V_ext.reshape(Skv, Hq * Dh).astype(jnp.bfloat16)

    def body(x_ref, wq_ref, wo_ref, k_ref, v_ref, out_ref,
             q_buf, acc_buf, ml_buf, acc_stage, ml_stage, va_ref,
             q_ssem, q_rsem, a_ssem, a_rsem, m_ssem, m_rsem):
        my = lax.axis_index("i")
        left = lax.rem(my + N_DEV - 1, N_DEV)
        right = lax.rem(my + 1, N_DEV)

        barrier_sem = pltpu.get_barrier_semaphore()
        for nbr in (left, right):
            pl.semaphore_signal(
                barrier_sem, inc=1,
                device_id=(nbr,), device_id_type=pl.DeviceIdType.MESH,
            )
        pl.semaphore_wait(barrier_sem, 2)

        for i in range(Hq):
            va_ref[i, :, 0:Dh] = v_ref[:, i * Dh:(i + 1) * Dh]
            va_ref[i, :, Dh:Dh + 1] = jnp.ones((Skv, 1), jnp.bfloat16)

        q0 = lax.dot_general(
            x_ref[:, :].astype(jnp.bfloat16),
            wq_ref[:, :].astype(jnp.bfloat16),
            (((1,), (0,)), ((), ())),
            preferred_element_type=jnp.float32,
        )
        q_buf[0] = (q0 * SCALE2).astype(jnp.bfloat16)

        q_rdmas = [None] * (N_DEV - 1)
        a_rdmas = {}
        m_rdmas = {}
        sends_unwaited = {}

        for h in range(N_DEV):
            if h > 0:
                q_rdmas[h - 1].wait_recv()
            if h < N_DEV - 1:
                rq = pltpu.make_async_remote_copy(
                    src_ref=q_buf.at[h], dst_ref=q_buf.at[h + 1],
                    send_sem=q_ssem.at[h], recv_sem=q_rsem.at[h],
                    device_id=(right,), device_id_type=pl.DeviceIdType.MESH,
                )
                rq.start()
                q_rdmas[h] = rq
                sends_unwaited[("q", h)] = rq

            for half in range(N_SPLIT):
                if h > 0:
                    a_rdmas[(h - 1, half)].wait_recv()
                    m_rdmas[(h - 1, half)].wait_recv()
                if h >= 2:
                    sends_unwaited.pop(("a", h - 2, half)).wait_send()
                    sends_unwaited.pop(("m", h - 2, half)).wait_send()
                rs = slice(half * rows, (half + 1) * rows)
                m_cols = []
                l_cols = []
                for i in range(Hq):
                    sl = slice(i * Dh, (i + 1) * Dh)
                    q_i = q_buf[h, rs, sl]
                    k_i = k_ref[:, sl]
                    s = lax.dot_general(
                        q_i, k_i, (((1,), (1,)), ((), ())),
                        preferred_element_type=jnp.float32,
                    )
                    mj = jnp.max(s, axis=1, keepdims=True)
                    if h == 0:
                        m_new = mj
                        p = jnp.exp2(s - m_new)
                        pv = lax.dot_general(
                            p.astype(jnp.bfloat16), va_ref[i],
                            (((1,), (0,)), ((), ())),
                            preferred_element_type=jnp.float32,
                        )
                        l_new = pv[:, Dh:Dh + 1]
                        acc_new = pv[:, 0:Dh]
                    else:
                        m_old = ml_buf[h - 1, rs, i:i + 1]
                        l_old = ml_buf[h - 1, rs, Hq + i:Hq + i + 1]
                        m_new = jnp.maximum(m_old, mj)
                        alpha = jnp.exp2(m_old - m_new)
                        p = jnp.exp2(s - m_new)
                        pv = lax.dot_general(
                            p.astype(jnp.bfloat16), va_ref[i],
                            (((1,), (0,)), ((), ())),
                            preferred_element_type=jnp.float32,
                        )
                        l_new = l_old * alpha + pv[:, Dh:Dh + 1]
                        acc_new = acc_buf[h - 1, rs, sl] * alpha + pv[:, 0:Dh]
                    m_cols.append(m_new)
                    l_cols.append(l_new)
                    acc_stage[h % 2, rs, sl] = acc_new.astype(jnp.bfloat16)

                ml_stage[h % 2, rs, :] = jnp.concatenate(m_cols + l_cols,
                                                         axis=1)

                r0 = half * rows
                ra = pltpu.make_async_remote_copy(
                    src_ref=acc_stage.at[h % 2, pl.ds(r0, rows)],
                    dst_ref=acc_buf.at[h, pl.ds(r0, rows)],
                    send_sem=a_ssem.at[h, half], recv_sem=a_rsem.at[h, half],
                    device_id=(right,), device_id_type=pl.DeviceIdType.MESH,
                )
                rm = pltpu.make_async_remote_copy(
                    src_ref=ml_stage.at[h % 2, pl.ds(r0, rows)],
                    dst_ref=ml_buf.at[h, pl.ds(r0, rows)],
                    send_sem=m_ssem.at[h, half], recv_sem=m_rsem.at[h, half],
                    device_id=(right,), device_id_type=pl.DeviceIdType.MESH,
                )
                ra.start()
                rm.start()
                a_rdmas[(h, half)] = ra
                m_rdmas[(h, half)] = rm
                sends_unwaited[("a", h, half)] = ra
                sends_unwaited[("m", h, half)] = rm

        wo_bf = wo_ref[:, :].astype(jnp.bfloat16)
        for half in range(N_SPLIT):
            rs = slice(half * rows, (half + 1) * rows)
            a_rdmas[(N_DEV - 1, half)].wait_recv()
            m_rdmas[(N_DEV - 1, half)].wait_recv()
            outs = []
            for i in range(Hq):
                sl = slice(i * Dh, (i + 1) * Dh)
                l_i = ml_buf[N_DEV - 1, rs, Hq + i:Hq + i + 1]
                outs.append(
                    (acc_buf[N_DEV - 1, rs, sl] / l_i).astype(jnp.bfloat16))
            o = jnp.concatenate(outs, axis=1)
            out_ref[rs, :] = lax.dot_general(
                o, wo_bf, (((1,), (0,)), ((), ())),
                preferred_element_type=jnp.float32,
            ).astype(jnp.bfloat16)
        for r in sends_unwaited.values():
            r.wait_send()

    out = pl.pallas_call(
        body,
        out_shape=jax.ShapeDtypeStruct((Sq, D), jnp.bfloat16),
        in_specs=[pl.BlockSpec(memory_space=pltpu.VMEM)] * 5,
        out_specs=pl.BlockSpec(memory_space=pltpu.VMEM),
        scratch_shapes=[
            pltpu.VMEM((N_DEV, Sq, Hq * Dh), jnp.bfloat16),
            pltpu.VMEM((N_DEV, Sq, Hq * Dh), jnp.bfloat16),
            pltpu.VMEM((N_DEV, Sq, 2 * Hq), jnp.float32),
            pltpu.VMEM((2, Sq, Hq * Dh), jnp.bfloat16),
            pltpu.VMEM((2, Sq, 2 * Hq), jnp.float32),
            pltpu.VMEM((Hq, Skv, Dh + 1), jnp.bfloat16),
            pltpu.SemaphoreType.DMA((N_DEV - 1,)),
            pltpu.SemaphoreType.DMA((N_DEV - 1,)),
            pltpu.SemaphoreType.DMA((N_DEV, N_SPLIT)),
            pltpu.SemaphoreType.DMA((N_DEV, N_SPLIT)),
            pltpu.SemaphoreType.DMA((N_DEV, N_SPLIT)),
            pltpu.SemaphoreType.DMA((N_DEV, N_SPLIT)),
        ],
        compiler_params=pltpu.CompilerParams(
            collective_id=0,
            vmem_limit_bytes=100 * 1024 * 1024,
        ),
    )(x2, Wq, Wo, K2, V2)
    return out.reshape(1, Sq, D)
